# baseline (speedup 1.0000x reference)
import functools

import jax
import jax.numpy as jnp
from jax import lax
from jax.experimental import pallas as pl
from jax.experimental.pallas import tpu as pltpu
from jax.experimental.pallas import tpu_sc as plsc

N_NODES = 10000
N_EDGES = 320000
FEATS = 128
NBASES = 4
NRELS = 1344
YCOLS = NBASES * FEATS

NCORES = 2
NSUB = 16
NWORK = NCORES * NSUB
CHUNK = 48
CPW = -(-N_EDGES // (CHUNK * NWORK))
E_PAD = CPW * NWORK * CHUNK
SLAB = 3 * CHUNK
N_PAD = 10240
ROWS_PER_SUB = N_PAD // NSUB


def _matmul_body(x_ref, b_ref, y_ref):
    y_ref[...] = jnp.dot(x_ref[...], b_ref[...],
                         preferred_element_type=jnp.float32).astype(jnp.bfloat16)


def _basis_transform(x, bmat):
    return pl.pallas_call(
        _matmul_body,
        out_shape=jax.ShapeDtypeStruct((N_NODES, YCOLS), jnp.bfloat16),
        grid=(10,),
        in_specs=[
            pl.BlockSpec((N_NODES // 10, FEATS), lambda i: (i, 0)),
            pl.BlockSpec((FEATS, YCOLS), lambda i: (0, 0)),
        ],
        out_specs=pl.BlockSpec((N_NODES // 10, YCOLS), lambda i: (i, 0)),
    )(x, bmat)


def _edge_body(idx_hbm, y_hbm, wc_hbm, zeros_hbm, out_hbm,
               ig0, ig1, rows0, rows1, cf0, cf1, msg0, msg1,
               dk0, dk1, wc_v, agg_sh,
               si0, si1, sy0, sy1, ss0, ss1):
    cid = lax.axis_index("c")
    sid = lax.axis_index("s")
    wid = cid * NSUB + sid

    ig = (ig0, ig1)
    rows = (rows0, rows1)
    cf = (cf0, cf1)
    msg = (msg0, msg1)
    dk = (dk0, dk1)
    si = (si0, si1)
    sy = (sy0, sy1)
    ss = (ss0, ss1)

    pltpu.sync_copy(wc_hbm, wc_v)
    pltpu.sync_copy(zeros_hbm.at[pl.ds(sid * ROWS_PER_SUB, ROWS_PER_SUB)],
                    agg_sh.at[pl.ds(sid * ROWS_PER_SUB, ROWS_PER_SUB)])
    plsc.subcore_barrier()

    chunk0 = wid * CPW
    lanes = lax.iota(jnp.int32, 16)

    def issue_idxgrp(kfirst, b):
        pltpu.async_copy(idx_hbm.at[pl.ds((chunk0 + kfirst) * SLAB, 4 * SLAB)],
                         ig[b], si[b])

    def wait_idxgrp(b):
        pltpu.make_async_copy(idx_hbm.at[pl.ds(0, 4 * SLAB)], ig[b], si[b]).wait()

    def issue_gather(buf, pos, b2):
        pltpu.async_copy(y_hbm.at[ig[buf].at[pl.ds(pos * SLAB, CHUNK)]],
                         rows[b2], sy[b2])

    def wait_rows(b2):
        pltpu.make_async_copy(y_hbm.at[ig[0].at[pl.ds(0, CHUNK)]],
                              rows[b2], sy[b2]).wait()

    def issue_scatter(b2):
        pltpu.async_copy(msg[b2], agg_sh.at[dk[b2]], ss[b2], add=True)

    def wait_scatter(b2):
        pltpu.make_async_copy(msg[b2], agg_sh.at[dk[b2]], ss[b2]).wait()

    def save_dst(pbuf, ppos, p2):
        for g in range(CHUNK // 16):
            dk[p2][pl.ds(g * 16, 16)] = ig[pbuf][pl.ds(ppos * SLAB + CHUNK + g * 16, 16)]

    def expand_cf(buf, pos, b2):
        for g in range(CHUNK // 16):
            evec = ig[buf][pl.ds(pos * SLAB + 2 * CHUNK + g * 16, 16)] * NBASES
            dst_ix = (g * 16 + lanes) * 16
            for bb in range(NBASES):
                cb = plsc.load_gather(wc_v, [evec + bb])
                plsc.store_scatter(cf[b2], [dst_ix + bb], cb)

    def compute(b2):
        def edge_body(e, inner):
            cvec = cf[b2][pl.ds(e * 16, 16)]
            cs = (cvec[0], cvec[1], cvec[2], cvec[3])
            for q in range(FEATS // 32):
                acc_lo = jnp.zeros((16,), jnp.float32)
                acc_hi = jnp.zeros((16,), jnp.float32)
                for bb in range(NBASES):
                    w = rows[b2][e, pl.ds(bb * (FEATS // 2) + q * 16, 16)]
                    wbf = plsc.bitcast(w, jnp.bfloat16)
                    lo, hi = plsc.unpack(wbf, format=plsc.PackFormat.INTERLEAVED)
                    acc_lo = acc_lo + cs[bb] * lo
                    acc_hi = acc_hi + cs[bb] * hi
                msg[b2][e, pl.ds(q * 32, 16)] = acc_lo
                msg[b2][e, pl.ds(q * 32 + 16, 16)] = acc_hi
            return inner

        lax.fori_loop(0, CHUNK, edge_body, 0, unroll=2)

    def stage(k, t):
        buf, pos = t // 4, t % 4
        pbuf, ppos = ((t - 1) % 8) // 4, ((t - 1) % 8) % 4
        a2 = (1 + t) % 2
        p2 = 1 - a2
        if pos == 0:
            wait_idxgrp(buf)
        issue_gather(buf, pos, a2)

        @pl.when(k >= 3)
        def _():
            wait_scatter(p2)

        save_dst(pbuf, ppos, p2)
        if pos == 0:
            @pl.when(k + 4 < CPW)
            def _():
                issue_idxgrp(k + 4, 1 - buf)

        expand_cf(buf, pos, a2)
        wait_rows(p2)
        compute(p2)
        issue_scatter(p2)

    pltpu.sync_copy(idx_hbm.at[pl.ds(chunk0 * SLAB, SLAB)],
                    ig[1].at[pl.ds(3 * SLAB, SLAB)])
    issue_idxgrp(1, 0)
    pltpu.sync_copy(idx_hbm.at[pl.ds(chunk0 * SLAB, CHUNK)], dk[1])
    pltpu.async_copy(y_hbm.at[dk[1]], rows[0], sy[0])
    expand_cf(1, 3, 0)

    def oct_body(j, carry):
        for t in range(8):
            stage(8 * j + 1 + t, t)
        return carry

    lax.fori_loop(0, (CPW - 1) // 8, oct_body, 0)

    wait_scatter(0)
    save_dst(1, 3, 0)
    wait_rows(0)
    compute(0)
    issue_scatter(0)
    wait_scatter(0)
    wait_scatter(1)
    plsc.subcore_barrier()
    pltpu.sync_copy(agg_sh.at[pl.ds(sid * ROWS_PER_SUB, ROWS_PER_SUB)],
                    out_hbm.at[pl.ds(cid * N_PAD + sid * ROWS_PER_SUB, ROWS_PER_SUB)])


@functools.cache
def _edge_kernel():
    return pl.kernel(
        _edge_body,
        out_type=jax.ShapeDtypeStruct((NCORES * N_PAD, FEATS), jnp.float32),
        mesh=plsc.VectorSubcoreMesh(core_axis_name="c", subcore_axis_name="s",
                                    num_cores=NCORES, num_subcores=NSUB),
        compiler_params=pltpu.CompilerParams(needs_layout_passes=False),
        scratch_types=[
            pltpu.VMEM((4 * SLAB,), jnp.int32),
            pltpu.VMEM((4 * SLAB,), jnp.int32),
            pltpu.VMEM((CHUNK, YCOLS // 2), jnp.int32),
            pltpu.VMEM((CHUNK, YCOLS // 2), jnp.int32),
            pltpu.VMEM((CHUNK * 16,), jnp.float32),
            pltpu.VMEM((CHUNK * 16,), jnp.float32),
            pltpu.VMEM((CHUNK, FEATS), jnp.float32),
            pltpu.VMEM((CHUNK, FEATS), jnp.float32),
            pltpu.VMEM((CHUNK,), jnp.int32),
            pltpu.VMEM((CHUNK,), jnp.int32),
            pltpu.VMEM((NRELS * NBASES,), jnp.float32),
            pltpu.VMEM_SHARED((N_PAD, FEATS), jnp.float32),
            pltpu.SemaphoreType.DMA,
            pltpu.SemaphoreType.DMA,
            pltpu.SemaphoreType.DMA,
            pltpu.SemaphoreType.DMA,
            pltpu.SemaphoreType.DMA,
            pltpu.SemaphoreType.DMA,
        ],
    )


def _epilogue_body(p_ref, x_ref, bias_ref, gam_ref, bet_ref, o_ref):
    h = p_ref[0, :N_NODES] + p_ref[1, :N_NODES] + bias_ref[...]
    h = jnp.maximum(h, 0.0) + x_ref[...]
    mean = jnp.mean(h, axis=0, keepdims=True)
    var = jnp.mean((h - mean) * (h - mean), axis=0, keepdims=True)
    hn = gam_ref[...] * (h - mean) * lax.rsqrt(var + 1e-5) + bet_ref[...]
    o_ref[...] = jnp.maximum(hn, 0.0)


def _epilogue(parts, x, bias, gamma, beta):
    return pl.pallas_call(
        _epilogue_body,
        out_shape=jax.ShapeDtypeStruct((N_NODES, FEATS), jnp.float32),
    )(parts, x, bias, gamma, beta)


def kernel(node_feats, edge_index, etype, basis, w_comp, h_bias, bn_gamma, bn_beta):
    x = node_feats.astype(jnp.float32)
    src = edge_index[0].astype(jnp.int32)
    dst = edge_index[1].astype(jnp.int32)
    et = etype.astype(jnp.int32)
    bmat = jnp.transpose(basis, (1, 0, 2)).reshape(FEATS, YCOLS)
    perm = []
    for b in range(NBASES):
        for q in range(FEATS // 32):
            for i in range(16):
                perm.extend([b * FEATS + q * 32 + i, b * FEATS + q * 32 + 16 + i])
    bmat = bmat[:, jnp.array(perm, jnp.int32)]
    wc_flat = w_comp.astype(jnp.float32).reshape(-1)

    pad = E_PAD - N_EDGES
    srcp = jnp.concatenate([src, jnp.zeros((pad,), jnp.int32)])
    dstp = jnp.concatenate([dst, jnp.full((pad,), N_NODES, jnp.int32)])
    etp = jnp.concatenate([et, jnp.zeros((pad,), jnp.int32)])
    idx_pack = (jnp.stack([srcp, dstp, etp], axis=0)
                .reshape(3, E_PAD // CHUNK, CHUNK)
                .transpose(1, 0, 2)
                .reshape(-1))

    y = lax.bitcast_convert_type(
        _basis_transform(x, bmat).reshape(N_NODES, YCOLS // 2, 2), jnp.int32)
    zeros = jnp.zeros((N_PAD, FEATS), jnp.float32)
    parts = _edge_kernel()(idx_pack, y, wc_flat, zeros)
    parts = parts.reshape(NCORES, N_PAD, FEATS)
    return _epilogue(parts, x, h_bias.reshape(1, FEATS),
                     bn_gamma.reshape(1, FEATS), bn_beta.reshape(1, FEATS))


# speedup vs baseline: 1.4444x; 1.4444x over previous
import functools

import jax
import jax.numpy as jnp
from jax import lax
from jax.experimental import pallas as pl
from jax.experimental.pallas import tpu as pltpu
from jax.experimental.pallas import tpu_sc as plsc

N_NODES = 10000
N_EDGES = 320000
FEATS = 128
NBASES = 4
NRELS = 1344
YCOLS = NBASES * FEATS

NCORES = 2
NSUB = 16
NWORK = NCORES * NSUB
CHUNK = 48
CPW = -(-N_EDGES // (CHUNK * NWORK))
E_PAD = CPW * NWORK * CHUNK
SLAB = 3 * CHUNK
N_PAD = 10240
ROWS_PER_SUB = N_PAD // NSUB


def _matmul_body(x_ref, b_ref, y_ref):
    r = jnp.dot(x_ref[...], b_ref[...],
                preferred_element_type=jnp.float32).astype(jnp.bfloat16)
    lo = lax.bitcast_convert_type(r[:, :YCOLS // 2], jnp.uint16).astype(jnp.uint32)
    hi = lax.bitcast_convert_type(r[:, YCOLS // 2:], jnp.uint16).astype(jnp.uint32)
    y_ref[...] = lax.bitcast_convert_type((hi << 16) | lo, jnp.int32)


def _basis_transform(x, bmat):
    return pl.pallas_call(
        _matmul_body,
        out_shape=jax.ShapeDtypeStruct((N_NODES, YCOLS // 2), jnp.int32),
        grid=(10,),
        in_specs=[
            pl.BlockSpec((N_NODES // 10, FEATS), lambda i: (i, 0)),
            pl.BlockSpec((FEATS, YCOLS), lambda i: (0, 0)),
        ],
        out_specs=pl.BlockSpec((N_NODES // 10, YCOLS // 2), lambda i: (i, 0)),
    )(x, bmat)


def _edge_body(src_hbm, dst_hbm, et_hbm, y_hbm, wc_hbm, zeros_hbm, out_hbm,
               sv0, sv1, dv0, dv1, ev0, ev1, rows0, rows1, cf0, cf1,
               msg0, msg1, dk0, dk1, wc_v, agg_sh,
               si0, si1, sy0, sy1, ss0, ss1):
    cid = lax.axis_index("c")
    sid = lax.axis_index("s")
    wid = cid * NSUB + sid

    sv = (sv0, sv1)
    dv = (dv0, dv1)
    ev = (ev0, ev1)
    rows = (rows0, rows1)
    cf = (cf0, cf1)
    msg = (msg0, msg1)
    dk = (dk0, dk1)
    si = (si0, si1)
    sy = (sy0, sy1)
    ss = (ss0, ss1)

    pltpu.sync_copy(wc_hbm, wc_v)
    pltpu.sync_copy(zeros_hbm.at[pl.ds(sid * ROWS_PER_SUB, ROWS_PER_SUB)],
                    agg_sh.at[pl.ds(sid * ROWS_PER_SUB, ROWS_PER_SUB)])
    plsc.subcore_barrier()

    chunk0 = wid * CPW
    lanes = lax.iota(jnp.int32, 16)

    def issue_idx(k, b):
        base = (chunk0 + k) * CHUNK
        pltpu.async_copy(src_hbm.at[pl.ds(base, CHUNK)], sv[b], si[b])
        pltpu.async_copy(dst_hbm.at[pl.ds(base, CHUNK)], dv[b], si[b])
        pltpu.async_copy(et_hbm.at[pl.ds(base, CHUNK)], ev[b], si[b])

    def wait_idx(b):
        pltpu.make_async_copy(src_hbm.at[pl.ds(0, CHUNK)], sv[b], si[b]).wait()
        pltpu.make_async_copy(dst_hbm.at[pl.ds(0, CHUNK)], dv[b], si[b]).wait()
        pltpu.make_async_copy(et_hbm.at[pl.ds(0, CHUNK)], ev[b], si[b]).wait()

    def issue_gather(b):
        pltpu.async_copy(y_hbm.at[sv[b]], rows[b], sy[b])

    def wait_rows(b):
        pltpu.make_async_copy(y_hbm.at[sv[b]], rows[b], sy[b]).wait()

    def issue_scatter(b):
        pltpu.async_copy(msg[b], agg_sh.at[dk[b]], ss[b], add=True)

    def wait_scatter(b):
        pltpu.make_async_copy(msg[b], agg_sh.at[dk[b]], ss[b]).wait()

    def save_dst(b):
        for g in range(CHUNK // 16):
            dk[b][pl.ds(g * 16, 16)] = dv[b][pl.ds(g * 16, 16)]

    def expand_cf(b):
        for g in range(CHUNK // 16):
            evec = ev[b][pl.ds(g * 16, 16)] * NBASES
            dst_ix = (g * 16 + lanes) * 16
            for bb in range(NBASES):
                cb = plsc.load_gather(wc_v, [evec + bb])
                plsc.store_scatter(cf[b], [dst_ix + bb], cb)

    def compute(b):
        def edge_body(e, inner):
            cvec = cf[b][pl.ds(e * 16, 16)]
            cs = (cvec[0], cvec[1], cvec[2], cvec[3])
            for q in range(FEATS // 32):
                acc_lo = jnp.zeros((16,), jnp.float32)
                acc_hi = jnp.zeros((16,), jnp.float32)
                for bb in range(NBASES):
                    w = rows[b][e, pl.ds(bb * (FEATS // 2) + q * 16, 16)]
                    wbf = plsc.bitcast(w, jnp.bfloat16)
                    lo, hi = plsc.unpack(wbf, format=plsc.PackFormat.INTERLEAVED)
                    acc_lo = acc_lo + cs[bb] * lo
                    acc_hi = acc_hi + cs[bb] * hi
                msg[b][e, pl.ds(q * 32, 16)] = acc_lo
                msg[b][e, pl.ds(q * 32 + 16, 16)] = acc_hi
            return inner

        lax.fori_loop(0, CHUNK, edge_body, 0, unroll=2)

    def stage(k, a):
        b = 1 - a
        wait_idx(a)
        issue_gather(a)

        @pl.when(k >= 3)
        def _():
            wait_scatter(b)

        save_dst(b)
        wait_rows(b)

        @pl.when(k + 1 < CPW)
        def _():
            issue_idx(k + 1, b)

        expand_cf(a)
        compute(b)
        issue_scatter(b)

    base00 = chunk0 * CHUNK
    pltpu.sync_copy(src_hbm.at[pl.ds(base00, CHUNK)], sv[0])
    pltpu.sync_copy(dst_hbm.at[pl.ds(base00, CHUNK)], dv[0])
    pltpu.sync_copy(et_hbm.at[pl.ds(base00, CHUNK)], ev[0])
    issue_gather(0)
    issue_idx(1, 1)
    expand_cf(0)

    def pair_body(j, carry):
        stage(2 * j + 1, 1)
        stage(2 * j + 2, 0)
        return carry

    lax.fori_loop(0, (CPW - 1) // 2, pair_body, 0)

    wait_scatter(0)
    save_dst(0)
    wait_rows(0)
    compute(0)
    issue_scatter(0)
    wait_scatter(0)
    wait_scatter(1)
    plsc.subcore_barrier()
    pltpu.sync_copy(agg_sh.at[pl.ds(sid * ROWS_PER_SUB, ROWS_PER_SUB)],
                    out_hbm.at[pl.ds(cid * N_PAD + sid * ROWS_PER_SUB, ROWS_PER_SUB)])


@functools.cache
def _edge_kernel():
    return pl.kernel(
        _edge_body,
        out_type=jax.ShapeDtypeStruct((NCORES * N_PAD, FEATS), jnp.float32),
        mesh=plsc.VectorSubcoreMesh(core_axis_name="c", subcore_axis_name="s",
                                    num_cores=NCORES, num_subcores=NSUB),
        compiler_params=pltpu.CompilerParams(needs_layout_passes=False),
        scratch_types=[
            pltpu.VMEM((CHUNK,), jnp.int32),
            pltpu.VMEM((CHUNK,), jnp.int32),
            pltpu.VMEM((CHUNK,), jnp.int32),
            pltpu.VMEM((CHUNK,), jnp.int32),
            pltpu.VMEM((CHUNK,), jnp.int32),
            pltpu.VMEM((CHUNK,), jnp.int32),
            pltpu.VMEM((CHUNK, YCOLS // 2), jnp.int32),
            pltpu.VMEM((CHUNK, YCOLS // 2), jnp.int32),
            pltpu.VMEM((CHUNK * 16,), jnp.float32),
            pltpu.VMEM((CHUNK * 16,), jnp.float32),
            pltpu.VMEM((CHUNK, FEATS), jnp.float32),
            pltpu.VMEM((CHUNK, FEATS), jnp.float32),
            pltpu.VMEM((CHUNK,), jnp.int32),
            pltpu.VMEM((CHUNK,), jnp.int32),
            pltpu.VMEM((NRELS * NBASES,), jnp.float32),
            pltpu.VMEM_SHARED((N_PAD, FEATS), jnp.float32),
            pltpu.SemaphoreType.DMA,
            pltpu.SemaphoreType.DMA,
            pltpu.SemaphoreType.DMA,
            pltpu.SemaphoreType.DMA,
            pltpu.SemaphoreType.DMA,
            pltpu.SemaphoreType.DMA,
        ],
    )


def _epilogue_body(p_ref, x_ref, bias_ref, gam_ref, bet_ref, o_ref):
    h = p_ref[0, :N_NODES] + p_ref[1, :N_NODES] + bias_ref[...]
    h = jnp.maximum(h, 0.0) + x_ref[...]
    mean = jnp.mean(h, axis=0, keepdims=True)
    var = jnp.mean((h - mean) * (h - mean), axis=0, keepdims=True)
    hn = gam_ref[...] * (h - mean) * lax.rsqrt(var + 1e-5) + bet_ref[...]
    o_ref[...] = jnp.maximum(hn, 0.0)


def _epilogue(parts, x, bias, gamma, beta):
    return pl.pallas_call(
        _epilogue_body,
        out_shape=jax.ShapeDtypeStruct((N_NODES, FEATS), jnp.float32),
    )(parts, x, bias, gamma, beta)


def kernel(node_feats, edge_index, etype, basis, w_comp, h_bias, bn_gamma, bn_beta):
    x = node_feats.astype(jnp.float32)
    src = edge_index[0].astype(jnp.int32)
    dst = edge_index[1].astype(jnp.int32)
    et = etype.astype(jnp.int32)
    ft = jnp.transpose(basis, (1, 0, 2)).reshape(FEATS, NBASES, NBASES, 2, 16)
    bmat = jnp.concatenate([ft[:, :, :, 0, :].reshape(FEATS, YCOLS // 2),
                            ft[:, :, :, 1, :].reshape(FEATS, YCOLS // 2)], axis=1)
    wc_flat = w_comp.astype(jnp.float32).reshape(-1)

    pad = E_PAD - N_EDGES
    srcp = jnp.concatenate([src, jnp.zeros((pad,), jnp.int32)])
    dstp = jnp.concatenate([dst, jnp.full((pad,), N_NODES, jnp.int32)])
    etp = jnp.concatenate([et, jnp.zeros((pad,), jnp.int32)])

    y = _basis_transform(x, bmat)
    zeros = jnp.zeros((N_PAD, FEATS), jnp.float32)
    parts = _edge_kernel()(srcp, dstp, etp, y, wc_flat, zeros)
    parts = parts.reshape(NCORES, N_PAD, FEATS)
    return _epilogue(parts, x, h_bias.reshape(1, FEATS),
                     bn_gamma.reshape(1, FEATS), bn_beta.reshape(1, FEATS))
